# Initial kernel scaffold; baseline (speedup 1.0000x reference)
import jax
import jax.numpy as jnp
from jax.experimental import pallas as pl


def kernel(log_probabilities, topk_log_probabilities, growing_beam):
    raise NotImplementedError("write your pallas kernel here")


# speedup vs baseline: 1.4116x; 1.4116x over previous
import functools

import jax
import jax.numpy as jnp
from jax.experimental import pallas as pl

_BEAM = 8
_EOS = 2
_NEG = -1e20
_ALPHA = 0.6
_STEP = 5


def _beam_step_kernel(lp_ref, bias_ref, gb_ref, vals_ref, scores_ref,
                      beam_out_ref, rows_ref, fin_ref, vocab: int):
    b = pl.program_id(0)

    x = lp_ref[...] + bias_ref[...]
    col = jax.lax.broadcasted_iota(jnp.int32, x.shape, 1)
    row = jax.lax.broadcasted_iota(jnp.int32, x.shape, 0)
    flat = row * vocab + col
    x = jnp.where(col == _EOS, _NEG, x)

    big = jnp.int32(2147483647)
    neg_inf = jnp.float32(-jnp.inf)

    vals = jnp.zeros((_BEAM,), jnp.float32)
    ids = jnp.zeros((_BEAM,), jnp.int32)
    k_iota = jax.lax.iota(jnp.int32, _BEAM)
    for i in range(_BEAM):
        m = jnp.max(x)
        idx = jnp.min(jnp.where(x == m, flat, big))
        vals = jnp.where(k_iota == i, m, vals)
        ids = jnp.where(k_iota == i, idx, ids)
        x = jnp.where(flat == idx, neg_inf, x)

    beam_ids = jnp.zeros((_BEAM,), jnp.int32)
    for r in range(1, _BEAM):
        beam_ids = beam_ids + jnp.where(ids >= r * vocab, 1, 0).astype(jnp.int32)
    token_ids = ids - beam_ids * vocab

    length_pen = ((5.0 + (_STEP + 1)) / 6.0) ** _ALPHA
    vals_ref[...] = vals.reshape(1, 1, _BEAM)
    scores_ref[...] = (vals / length_pen).reshape(1, 1, _BEAM)
    rows_ref[...] = (beam_ids + b * _BEAM).reshape(1, 1, _BEAM)
    fin_ref[...] = (token_ids == _EOS).reshape(1, 1, _BEAM)

    onehot = (beam_ids[:, None] == k_iota[None, :]).astype(jnp.float32)
    hist = jax.lax.dot_general(
        onehot, gb_ref[...].astype(jnp.float32),
        (((1,), (0,)), ((), ())), preferred_element_type=jnp.float32)
    beam_out_ref[:, :12] = hist.astype(jnp.int32)
    beam_out_ref[:, 12:13] = token_ids[:, None]


def kernel(log_probabilities, topk_log_probabilities, growing_beam):
    n_rows, vocab = log_probabilities.shape
    batch = n_rows // _BEAM
    hist = growing_beam.shape[1]

    grid = (batch,)
    out_shapes = (
        jax.ShapeDtypeStruct((batch, 1, _BEAM), jnp.float32),
        jax.ShapeDtypeStruct((batch, 1, _BEAM), jnp.float32),
        jax.ShapeDtypeStruct((n_rows, hist + 1), jnp.int32),
        jax.ShapeDtypeStruct((batch, 1, _BEAM), jnp.int32),
        jax.ShapeDtypeStruct((batch, 1, _BEAM), jnp.bool_),
    )
    small = pl.BlockSpec((1, 1, _BEAM), lambda b: (b, 0, 0))
    outs = pl.pallas_call(
        functools.partial(_beam_step_kernel, vocab=vocab),
        grid=grid,
        in_specs=[
            pl.BlockSpec((_BEAM, vocab), lambda b: (b, 0)),
            pl.BlockSpec((_BEAM, 1), lambda b: (b, 0)),
            pl.BlockSpec((_BEAM, hist), lambda b: (b, 0)),
        ],
        out_specs=(small, small,
                   pl.BlockSpec((_BEAM, hist + 1), lambda b: (b, 0)),
                   small, small),
        out_shape=out_shapes,
    )(log_probabilities, topk_log_probabilities.reshape(n_rows, 1),
      growing_beam)

    topk_log_probs, topk_scores, new_growing_beam, rows3, fin3 = outs
    return (topk_log_probs.reshape(batch, _BEAM),
            topk_scores.reshape(batch, _BEAM),
            new_growing_beam,
            rows3.reshape(-1),
            fin3.reshape(batch, _BEAM))
